# Initial kernel scaffold; baseline (speedup 1.0000x reference)
import functools

import jax
import jax.numpy as jnp
from jax import lax
from jax.experimental import pallas as pl
from jax.experimental.pallas import tpu as pltpu
from jax.experimental.pallas import tpu_sc as plsc

N = 16384
W = 50
D = 32
NC = 2
NS = 16
NW = NC * NS
UPT = N // NW
C = 16
NCHUNK = UPT // C


def _body(state_hbm, pos_hbm, neg_hbm, out_hbm, idx_v, sidx_v, rows_v, out_v, sem):
    wid = lax.axis_index("s") * NC + lax.axis_index("c")
    tile_base = wid * UPT

    def chunk_body(ci, carry):
        u0 = tile_base + ci * C
        pltpu.sync_copy(state_hbm.at[pl.ds(u0, C)], idx_v)
        for u in range(C):
            for t in range(2):
                for k0 in (0, 16, 32, W - 16):
                    sidx_v[u, t, pl.ds(k0, 16)] = idx_v[u, t, pl.ds(k0, 16)] + 1
        copies = []
        for u in range(C):
            copies.append(
                pltpu.async_copy(
                    pos_hbm.at[sidx_v.at[u, 0]], rows_v.at[pl.ds(u * 2 * W, W)], sem
                )
            )
            copies.append(
                pltpu.async_copy(
                    neg_hbm.at[sidx_v.at[u, 1]], rows_v.at[pl.ds(u * 2 * W + W, W)], sem
                )
            )
        for cp in copies:
            cp.wait()
        for u in range(C):
            def rbody(r, accs):
                a, b = accs
                row = u * 2 * W + r
                return (a + rows_v[row, pl.ds(0, 16)], b + rows_v[row, pl.ds(16, 16)])

            a, b = lax.fori_loop(
                0, 2 * W, rbody,
                (jnp.zeros((16,), jnp.float32), jnp.zeros((16,), jnp.float32)),
            )
            out_v[u, pl.ds(0, 16)] = a * (1.0 / W)
            out_v[u, pl.ds(16, 16)] = b * (1.0 / W)
        pltpu.sync_copy(out_v, out_hbm.at[pl.ds(u0, C)])
        return carry

    lax.fori_loop(0, NCHUNK, chunk_body, 0)


_user_model_sc = functools.partial(
    pl.kernel,
    out_type=jax.ShapeDtypeStruct((N, D), jnp.float32),
    mesh=plsc.VectorSubcoreMesh(core_axis_name="c", subcore_axis_name="s"),
    scratch_types=[
        pltpu.VMEM((C, 2, W), jnp.int32),
        pltpu.VMEM((C, 2, W), jnp.int32),
        pltpu.VMEM((C * 2 * W, D), jnp.float32),
        pltpu.VMEM((C, D), jnp.float32),
        pltpu.SemaphoreType.DMA,
    ],
)(_body)


def kernel(state, item_pos_emb, item_neg_emb):
    return _user_model_sc(state, item_pos_emb, item_neg_emb)


# speedup vs baseline: 2.3980x; 2.3980x over previous
import functools

import jax
import jax.numpy as jnp
from jax import lax
from jax.experimental import pallas as pl
from jax.experimental.pallas import tpu as pltpu
from jax.experimental.pallas import tpu_sc as plsc

N = 16384
W = 50
D = 32
NC = 2
NS = 16
NW = NC * NS
UPT = N // NW
C = 16
NCHUNK = UPT // C


def _body(state_hbm, pos_hbm, neg_hbm, out_hbm, idx_v, sidx_v, rows_v, out_v, sem):
    wid = lax.axis_index("s") * NC + lax.axis_index("c")
    tile_base = wid * UPT

    def chunk_body(ci, carry):
        u0 = tile_base + ci * C
        pltpu.sync_copy(state_hbm.at[pl.ds(u0, C)], idx_v)
        for u in range(C):
            for t in range(2):
                for k0 in (0, 16, 32, W - 16):
                    sidx_v[u, t, pl.ds(k0, 16)] = idx_v[u, t, pl.ds(k0, 16)] + 1
        copies = []
        for u in range(C):
            copies.append(
                pltpu.async_copy(
                    pos_hbm.at[sidx_v.at[u, 0]], rows_v.at[pl.ds(u * 2 * W, W)], sem
                )
            )
            copies.append(
                pltpu.async_copy(
                    neg_hbm.at[sidx_v.at[u, 1]], rows_v.at[pl.ds(u * 2 * W + W, W)], sem
                )
            )
        for cp in copies:
            cp.wait()
        for u in range(C):
            def rbody(r, accs):
                a, b = accs
                row = u * 2 * W + r
                return (a + rows_v[row, pl.ds(0, 16)], b + rows_v[row, pl.ds(16, 16)])

            a, b = lax.fori_loop(
                0, 2 * W, rbody,
                (jnp.zeros((16,), jnp.float32), jnp.zeros((16,), jnp.float32)),
            )
            out_v[u, pl.ds(0, 16)] = a * (1.0 / W)
            out_v[u, pl.ds(16, 16)] = b * (1.0 / W)
        pltpu.sync_copy(out_v, out_hbm.at[pl.ds(u0, C)])
        return carry

    lax.fori_loop(0, NCHUNK, chunk_body, 0)


_user_model_sc = functools.partial(
    pl.kernel,
    out_type=jax.ShapeDtypeStruct((N, D), jnp.float32),
    mesh=plsc.VectorSubcoreMesh(core_axis_name="c", subcore_axis_name="s"),
    scratch_types=[
        pltpu.VMEM((C, 2, W), jnp.int32),
        pltpu.VMEM((C, 2, W), jnp.int32),
        pltpu.VMEM((C * 2 * W, D), jnp.float32),
        pltpu.VMEM((C, D), jnp.float32),
        pltpu.SemaphoreType.DMA,
    ],
    compiler_params=pltpu.CompilerParams(use_tc_tiling_on_sc=False),
)(_body)


def kernel(state, item_pos_emb, item_neg_emb):
    return _user_model_sc(state, item_pos_emb, item_neg_emb)


# speedup vs baseline: 2.7997x; 1.1675x over previous
import functools

import jax
import jax.numpy as jnp
from jax import lax
from jax.experimental import pallas as pl
from jax.experimental.pallas import tpu as pltpu
from jax.experimental.pallas import tpu_sc as plsc

N = 16384
W = 50
D = 32
NC = 2
NS = 16
NW = NC * NS
UPT = N // NW
C = 8
R = C * 2 * W
NCHUNK = UPT // C
HALF = 1.0 / W


def _body(state_hbm, pos_hbm, neg_hbm, out_hbm,
          state_v, sidx0, sidx1, rows0, rows1, out0, out1,
          gsem0, gsem1, osem0, osem1):
    wid = lax.axis_index("s") * NC + lax.axis_index("c")
    tile_base = wid * UPT

    pltpu.sync_copy(state_hbm.at[pl.ds(tile_base, UPT)], state_v)

    def prep(ci, sidx):
        cbase = ci * C
        for u in range(C):
            for t in range(2):
                for k0 in (0, 16, 32, W - 16):
                    sidx[u, t, pl.ds(k0, 16)] = (
                        state_v[cbase + u, t, pl.ds(k0, 16)] + 1
                    )

    def fire(sidx, rows, sem):
        for u in range(C):
            pltpu.async_copy(
                pos_hbm.at[sidx.at[u, 0]], rows.at[pl.ds(u * 2 * W, W)], sem
            )
            pltpu.async_copy(
                neg_hbm.at[sidx.at[u, 1]], rows.at[pl.ds(u * 2 * W + W, W)], sem
            )

    def drain_gathers(rows, sem):
        pltpu.make_async_copy(pos_hbm.at[pl.ds(0, R)], rows, sem).wait()

    def wait_out(out, osem):
        pltpu.make_async_copy(out, out_hbm.at[pl.ds(0, C)], osem).wait()

    def reduce_store(ci, rows, out, osem):
        for u in range(C):
            def rbody(r, accs):
                base = u * 2 * W + r * 4
                a0, b0, a1, b1, a2, b2, a3, b3 = accs
                return (
                    a0 + rows[base, pl.ds(0, 16)],
                    b0 + rows[base, pl.ds(16, 16)],
                    a1 + rows[base + 1, pl.ds(0, 16)],
                    b1 + rows[base + 1, pl.ds(16, 16)],
                    a2 + rows[base + 2, pl.ds(0, 16)],
                    b2 + rows[base + 2, pl.ds(16, 16)],
                    a3 + rows[base + 3, pl.ds(0, 16)],
                    b3 + rows[base + 3, pl.ds(16, 16)],
                )

            z = jnp.zeros((16,), jnp.float32)
            a0, b0, a1, b1, a2, b2, a3, b3 = lax.fori_loop(
                0, 2 * W // 4, rbody, (z, z, z, z, z, z, z, z)
            )
            out[u, pl.ds(0, 16)] = ((a0 + a1) + (a2 + a3)) * HALF
            out[u, pl.ds(16, 16)] = ((b0 + b1) + (b2 + b3)) * HALF
        pltpu.async_copy(out, out_hbm.at[pl.ds(tile_base + ci * C, C)], osem)

    prep(0, sidx0)
    fire(sidx0, rows0, gsem0)

    def loop_body(i, carry):
        ci0 = 2 * i
        ci1 = 2 * i + 1
        prep(ci1, sidx1)
        drain_gathers(rows0, gsem0)
        fire(sidx1, rows1, gsem1)
        pl.when(i >= 1)(lambda: wait_out(out0, osem0))
        reduce_store(ci0, rows0, out0, osem0)
        pl.when(i < NCHUNK // 2 - 1)(lambda: prep(ci0 + 2, sidx0))
        drain_gathers(rows1, gsem1)
        pl.when(i < NCHUNK // 2 - 1)(lambda: fire(sidx0, rows0, gsem0))
        pl.when(i >= 1)(lambda: wait_out(out1, osem1))
        reduce_store(ci1, rows1, out1, osem1)
        return carry

    lax.fori_loop(0, NCHUNK // 2, loop_body, 0)
    wait_out(out0, osem0)
    wait_out(out1, osem1)


_user_model_sc = functools.partial(
    pl.kernel,
    out_type=jax.ShapeDtypeStruct((N, D), jnp.float32),
    mesh=plsc.VectorSubcoreMesh(core_axis_name="c", subcore_axis_name="s"),
    scratch_types=[
        pltpu.VMEM((UPT, 2, W), jnp.int32),
        pltpu.VMEM((C, 2, W), jnp.int32),
        pltpu.VMEM((C, 2, W), jnp.int32),
        pltpu.VMEM((R, D), jnp.float32),
        pltpu.VMEM((R, D), jnp.float32),
        pltpu.VMEM((C, D), jnp.float32),
        pltpu.VMEM((C, D), jnp.float32),
        pltpu.SemaphoreType.DMA,
        pltpu.SemaphoreType.DMA,
        pltpu.SemaphoreType.DMA,
        pltpu.SemaphoreType.DMA,
    ],
    compiler_params=pltpu.CompilerParams(use_tc_tiling_on_sc=False),
)(_body)


def kernel(state, item_pos_emb, item_neg_emb):
    return _user_model_sc(state, item_pos_emb, item_neg_emb)


# speedup vs baseline: 2.8553x; 1.0199x over previous
import functools

import jax
import jax.numpy as jnp
from jax import lax
from jax.experimental import pallas as pl
from jax.experimental.pallas import tpu as pltpu
from jax.experimental.pallas import tpu_sc as plsc

N = 16384
W = 50
D = 32
NC = 2
NS = 16
NW = NC * NS
UPT = N // NW
C = 16
CW = C * W
R = 2 * CW
NCHUNK = UPT // C
INV_W = 1.0 / W


def _body(state_hbm, pos_hbm, neg_hbm, out_hbm,
          idx0, idx1, pc0, pc1, nc0, nc1, rows0, rows1, out0, out1,
          isem0, isem1, gsem0, gsem1, osem0, osem1):
    wid = lax.axis_index("s") * NC + lax.axis_index("c")
    tile_base = wid * UPT

    def load_idx(ci, idx, isem):
        pltpu.async_copy(state_hbm.at[pl.ds(tile_base + ci * C, C)], idx, isem)

    def wait_idx(idx, isem):
        pltpu.make_async_copy(state_hbm.at[pl.ds(0, C)], idx, isem).wait()

    def build_cidx(idx, pc, nc):
        for i in range(CW // 16):
            j = lax.iota(jnp.int32, 16) + (i * 16)
            u = lax.shift_right_logical(j * 1311, 16)
            w = j - u * W
            t0 = u * 0
            pc[pl.ds(i * 16, 16)] = plsc.load_gather(idx, [u, t0, w]) + 1
            nc[pl.ds(i * 16, 16)] = plsc.load_gather(idx, [u, t0 + 1, w]) + 1

    def fire(pc, nc, rows, sem):
        pltpu.async_copy(pos_hbm.at[pc], rows.at[pl.ds(0, CW)], sem)
        pltpu.async_copy(neg_hbm.at[nc], rows.at[pl.ds(CW, CW)], sem)

    def drain_gathers(rows, sem):
        pltpu.make_async_copy(pos_hbm.at[pl.ds(0, R)], rows, sem).wait()

    def wait_out(out, osem):
        pltpu.make_async_copy(out, out_hbm.at[pl.ds(0, C)], osem).wait()

    def reduce_store(ci, rows, out, osem):
        for u in range(C):
            def rbody(r, accs):
                base = u * W + r * 2
                a0, b0, a1, b1, a2, b2, a3, b3 = accs
                return (
                    a0 + rows[base, pl.ds(0, 16)],
                    b0 + rows[base, pl.ds(16, 16)],
                    a1 + rows[base + 1, pl.ds(0, 16)],
                    b1 + rows[base + 1, pl.ds(16, 16)],
                    a2 + rows[CW + base, pl.ds(0, 16)],
                    b2 + rows[CW + base, pl.ds(16, 16)],
                    a3 + rows[CW + base + 1, pl.ds(0, 16)],
                    b3 + rows[CW + base + 1, pl.ds(16, 16)],
                )

            z = jnp.zeros((16,), jnp.float32)
            a0, b0, a1, b1, a2, b2, a3, b3 = lax.fori_loop(
                0, W // 2, rbody, (z, z, z, z, z, z, z, z)
            )
            out[u, pl.ds(0, 16)] = ((a0 + a1) + (a2 + a3)) * INV_W
            out[u, pl.ds(16, 16)] = ((b0 + b1) + (b2 + b3)) * INV_W
        pltpu.async_copy(out, out_hbm.at[pl.ds(tile_base + ci * C, C)], osem)

    load_idx(0, idx0, isem0)
    wait_idx(idx0, isem0)
    build_cidx(idx0, pc0, nc0)
    fire(pc0, nc0, rows0, gsem0)
    load_idx(1, idx1, isem1)

    def loop_body(i, carry):
        ci0 = 2 * i
        ci1 = 2 * i + 1
        wait_idx(idx1, isem1)
        build_cidx(idx1, pc1, nc1)
        drain_gathers(rows0, gsem0)
        fire(pc1, nc1, rows1, gsem1)
        pl.when(i < NCHUNK // 2 - 1)(lambda: load_idx(ci0 + 2, idx0, isem0))
        pl.when(i >= 1)(lambda: wait_out(out0, osem0))
        reduce_store(ci0, rows0, out0, osem0)
        def prep_next():
            wait_idx(idx0, isem0)
            build_cidx(idx0, pc0, nc0)
        pl.when(i < NCHUNK // 2 - 1)(prep_next)
        drain_gathers(rows1, gsem1)
        pl.when(i < NCHUNK // 2 - 1)(lambda: fire(pc0, nc0, rows0, gsem0))
        pl.when(i < NCHUNK // 2 - 1)(lambda: load_idx(ci1 + 2, idx1, isem1))
        pl.when(i >= 1)(lambda: wait_out(out1, osem1))
        reduce_store(ci1, rows1, out1, osem1)
        return carry

    lax.fori_loop(0, NCHUNK // 2, loop_body, 0)
    wait_out(out0, osem0)
    wait_out(out1, osem1)


_user_model_sc = functools.partial(
    pl.kernel,
    out_type=jax.ShapeDtypeStruct((N, D), jnp.float32),
    mesh=plsc.VectorSubcoreMesh(core_axis_name="c", subcore_axis_name="s"),
    scratch_types=[
        pltpu.VMEM((C, 2, W), jnp.int32),
        pltpu.VMEM((C, 2, W), jnp.int32),
        pltpu.VMEM((CW,), jnp.int32),
        pltpu.VMEM((CW,), jnp.int32),
        pltpu.VMEM((CW,), jnp.int32),
        pltpu.VMEM((CW,), jnp.int32),
        pltpu.VMEM((R, D), jnp.float32),
        pltpu.VMEM((R, D), jnp.float32),
        pltpu.VMEM((C, D), jnp.float32),
        pltpu.VMEM((C, D), jnp.float32),
        pltpu.SemaphoreType.DMA,
        pltpu.SemaphoreType.DMA,
        pltpu.SemaphoreType.DMA,
        pltpu.SemaphoreType.DMA,
        pltpu.SemaphoreType.DMA,
        pltpu.SemaphoreType.DMA,
    ],
    compiler_params=pltpu.CompilerParams(
        use_tc_tiling_on_sc=False, needs_layout_passes=False
    ),
)(_body)


def kernel(state, item_pos_emb, item_neg_emb):
    return _user_model_sc(state, item_pos_emb, item_neg_emb)


# speedup vs baseline: 2.8587x; 1.0012x over previous
import functools

import jax
import jax.numpy as jnp
from jax import lax
from jax.experimental import pallas as pl
from jax.experimental.pallas import tpu as pltpu
from jax.experimental.pallas import tpu_sc as plsc

N = 16384
W = 50
D = 32
NC = 2
NS = 16
NW = NC * NS
UPT = N // NW
C = 8
CW = C * W
NBLK = UPT // 128
INV_W = 1.0 / W
MAGIC = 1311


def _body(s4_hbm, pos_hbm, neg_hbm, out4_hbm,
          svb0, svb1, pc0, pc1, nc0, nc1, rows0, rows1, fm0, fm1,
          gsem0, gsem1, fsem0, fsem1, ssem):
    wid = lax.axis_index("s") * NC + lax.axis_index("c")
    ub0 = wid * NBLK

    iota16 = lax.iota(jnp.int32, 16)
    a_lo = lax.shift_right_logical(iota16, 3)
    r_v = iota16 & 7

    def load_state(b, svb):
        pltpu.async_copy(s4_hbm.at[:, ub0 + b], svb, ssem)

    def wait_state(svb):
        pltpu.make_async_copy(s4_hbm.at[:, 0], svb, ssem).wait()

    def build(svb, pc, nc, uc0):
        for i in range(CW // 16):
            j = iota16 + (i * 16)
            uu = lax.shift_right_logical(j * MAGIC, 16)
            w = j - uu * W
            t0 = uu * 0
            ucv = uu + uc0
            pc[pl.ds(i * 16, 16)] = plsc.load_gather(svb, [w, t0, ucv]) + 1
            nc[pl.ds(i * 16, 16)] = plsc.load_gather(svb, [w, t0 + 1, ucv]) + 1

    def fire(pc, nc, rows, sem):
        pltpu.async_copy(pos_hbm.at[pc], rows.at[pl.ds(0, CW)], sem)
        pltpu.async_copy(neg_hbm.at[nc], rows.at[pl.ds(CW, CW)], sem)

    def drain_gathers(rows, sem):
        pltpu.make_async_copy(pos_hbm.at[pl.ds(0, 2 * CW)], rows, sem).wait()

    def reduce_scatter(rows, fm, uc0):
        for uu in range(C):
            def rbody(r, accs):
                base = uu * W + r * 2
                a0, b0, a1, b1, a2, b2, a3, b3 = accs
                return (
                    a0 + rows[base, pl.ds(0, 16)],
                    b0 + rows[base, pl.ds(16, 16)],
                    a1 + rows[base + 1, pl.ds(0, 16)],
                    b1 + rows[base + 1, pl.ds(16, 16)],
                    a2 + rows[CW + base, pl.ds(0, 16)],
                    b2 + rows[CW + base, pl.ds(16, 16)],
                    a3 + rows[CW + base + 1, pl.ds(0, 16)],
                    b3 + rows[CW + base + 1, pl.ds(16, 16)],
                )

            z = jnp.zeros((16,), jnp.float32)
            a0, b0, a1, b1, a2, b2, a3, b3 = lax.fori_loop(
                0, W // 2, rbody, (z, z, z, z, z, z, z, z)
            )
            lo = ((a0 + a1) + (a2 + a3)) * INV_W
            hi = ((b0 + b1) + (b2 + b3)) * INV_W
            c_spl = iota16 * 0 + (uc0 + uu)
            plsc.store_scatter(fm, [a_lo, r_v, c_spl], lo)
            plsc.store_scatter(fm, [a_lo + 2, r_v, c_spl], hi)

    def flush(fm, b, fsem):
        pltpu.async_copy(fm, out4_hbm.at[:, ub0 + b], fsem)

    def wait_flush(fm, fsem):
        pltpu.make_async_copy(fm, out4_hbm.at[:, 0], fsem).wait()

    load_state(0, svb0)
    wait_state(svb0)
    build(svb0, pc0, nc0, 0)
    fire(pc0, nc0, rows0, gsem0)
    load_state(1, svb1)

    svb = (svb0, svb1)
    fm = (fm0, fm1)
    fsem = (fsem0, fsem1)

    for b in range(NBLK):
        p = b & 1
        svb_q = svb[b & 1]
        fm_p = fm[p]

        def ibody(ii, carry):
            uc0_0 = ii * 16
            uc0_1 = ii * 16 + 8
            build(svb_q, pc1, nc1, uc0_1)
            drain_gathers(rows0, gsem0)
            fire(pc1, nc1, rows1, gsem1)
            reduce_scatter(rows0, fm_p, uc0_0)
            pl.when(ii < 7)(lambda: build(svb_q, pc0, nc0, uc0_0 + 16))
            drain_gathers(rows1, gsem1)
            pl.when(ii < 7)(lambda: fire(pc0, nc0, rows0, gsem0))
            reduce_scatter(rows1, fm_p, uc0_1)
            return carry

        if b >= 2:
            wait_flush(fm_p, fsem[p])
        lax.fori_loop(0, 8, ibody, 0)
        flush(fm_p, b, fsem[p])
        if b < NBLK - 1:
            wait_state(svb[(b + 1) & 1])
            if b < NBLK - 2:
                load_state(b + 2, svb[b & 1])
            build(svb[(b + 1) & 1], pc0, nc0, 0)
            fire(pc0, nc0, rows0, gsem0)

    wait_flush(fm0, fsem0)
    wait_flush(fm1, fsem1)


_user_model_sc = functools.partial(
    pl.kernel,
    out_type=jax.ShapeDtypeStruct((4, 128, 8, 128), jnp.float32),
    mesh=plsc.VectorSubcoreMesh(core_axis_name="c", subcore_axis_name="s"),
    scratch_types=[
        pltpu.VMEM((W, 2, 128), jnp.int32),
        pltpu.VMEM((W, 2, 128), jnp.int32),
        pltpu.VMEM((CW,), jnp.int32),
        pltpu.VMEM((CW,), jnp.int32),
        pltpu.VMEM((CW,), jnp.int32),
        pltpu.VMEM((CW,), jnp.int32),
        pltpu.VMEM((2 * CW, D), jnp.float32),
        pltpu.VMEM((2 * CW, D), jnp.float32),
        pltpu.VMEM((4, 8, 128), jnp.float32),
        pltpu.VMEM((4, 8, 128), jnp.float32),
        pltpu.SemaphoreType.DMA,
        pltpu.SemaphoreType.DMA,
        pltpu.SemaphoreType.DMA,
        pltpu.SemaphoreType.DMA,
        pltpu.SemaphoreType.DMA,
    ],
    compiler_params=pltpu.CompilerParams(
        use_tc_tiling_on_sc=False, needs_layout_passes=False
    ),
)(_body)


def kernel(state, item_pos_emb, item_neg_emb):
    s4 = state.transpose(2, 1, 0).reshape(W, 2, 128, 128).transpose(0, 2, 1, 3)
    out4 = _user_model_sc(s4, item_pos_emb, item_neg_emb)
    return out4.transpose(1, 3, 0, 2).reshape(N, D)
